# baseline (speedup 1.0000x reference)
import functools

import jax
import jax.numpy as jnp
from jax import lax
from jax.experimental import pallas as pl
from jax.experimental.pallas import tpu as pltpu
from jax.experimental.pallas import tpu_sc as plsc

N_TOK = 32768
D = 768
LANES = 16
NC = 2
NS = 16
NW = NC * NS
TPW = N_TOK // NW
C = 32
NCHUNK = TPW // C
NBUF = 2


def _affine_gate(x, idx, gb):
    mesh = plsc.VectorSubcoreMesh(core_axis_name="c", subcore_axis_name="s")

    scratch = [
        pltpu.VMEM((NBUF, C, D), jnp.float32),
        pltpu.VMEM((NBUF, C, D), jnp.int32),
        pltpu.VMEM((NBUF, C), jnp.int32),
    ]
    scratch += [pltpu.SemaphoreType.DMA] * (4 * NBUF)

    @functools.partial(
        pl.kernel,
        mesh=mesh,
        out_type=jax.ShapeDtypeStruct((N_TOK, D), jnp.float32),
        scratch_types=scratch,
    )
    def k(x_hbm, idx_hbm, gb_hbm, out_hbm, xb, gbb, ib, *sems):
        s_x = sems[0:NBUF]
        s_i = sems[NBUF:2 * NBUF]
        s_g = sems[2 * NBUF:3 * NBUF]
        s_o = sems[3 * NBUF:4 * NBUF]
        wid = lax.axis_index("s") * NC + lax.axis_index("c")
        wbase = wid * TPW

        def start_in(b, ci):
            base = wbase + ci * C
            pltpu.make_async_copy(
                x_hbm.at[pl.ds(base, C)], xb.at[b], s_x[b]).start()
            pltpu.make_async_copy(
                idx_hbm.at[pl.ds(base, C)], ib.at[b], s_i[b]).start()

        def start_gather(b):
            pltpu.make_async_copy(
                idx_hbm.at[pl.ds(0, C)], ib.at[b], s_i[b]).wait()
            pltpu.make_async_copy(gb_hbm.at[ib.at[b]], gbb.at[b], s_g[b]).start()

        def wait_in(b):
            pltpu.make_async_copy(
                x_hbm.at[pl.ds(0, C)], xb.at[b], s_x[b]).wait()
            pltpu.make_async_copy(
                gb_hbm.at[ib.at[b]], gbb.at[b], s_g[b]).wait()

        def start_out(b, ci):
            base = wbase + ci * C
            pltpu.make_async_copy(
                xb.at[b], out_hbm.at[pl.ds(base, C)], s_o[b]).start()

        def wait_out(b):
            pltpu.make_async_copy(
                xb.at[b], out_hbm.at[pl.ds(0, C)], s_o[b]).wait()

        def compute(b):
            def tok_body(t, tc):
                for d in range(D // LANES):
                    sl = pl.ds(d * LANES, LANES)
                    w = gbb[b, t, sl]
                    g = lax.bitcast_convert_type(w & jnp.int32(-65536), jnp.float32)
                    bta = lax.bitcast_convert_type(lax.shift_left(w, 16), jnp.float32)
                    xb[b, t, sl] = xb[b, t, sl] * g + bta
                return tc

            lax.fori_loop(0, C, tok_body, 0)

        for b in range(NBUF):
            start_in(b, b)
            start_gather(b)

        def group_body(g0, carry):
            for b in range(NBUF):
                ci = g0 * NBUF + b
                wait_in(b)
                compute(b)
                start_out(b, ci)
                bp = (b - 1) % NBUF
                cip = ci - 1 + NBUF

                @pl.when(jnp.logical_and(ci >= 1, cip < NCHUNK))
                def _():
                    wait_out(bp)
                    start_in(bp, cip)
                    start_gather(bp)

            return carry

        lax.fori_loop(0, NCHUNK // NBUF, group_body, 0)

        for b in range(NBUF):
            wait_out(b)

    return k(x, idx, gb)


def kernel(x, group, gamma, beta):
    idx = group.reshape(-1)
    gbits = jax.lax.bitcast_convert_type(
        gamma.astype(jnp.bfloat16), jnp.uint16).astype(jnp.uint32)
    bbits = jax.lax.bitcast_convert_type(
        beta.astype(jnp.bfloat16), jnp.uint16).astype(jnp.uint32)
    gb = jax.lax.bitcast_convert_type((gbits << 16) | bbits, jnp.int32)
    return _affine_gate(x, idx, gb)


# speedup vs baseline: 2.3770x; 2.3770x over previous
import functools

import jax
import jax.numpy as jnp
from jax import lax
from jax.experimental import pallas as pl
from jax.experimental.pallas import tpu as pltpu
from jax.experimental.pallas import tpu_sc as plsc

N_TOK = 32768
D = 768
LANES = 16
NC = 2
NS = 16
NW = NC * NS
TPW = N_TOK // NW
C = 64
NCHUNK = TPW // C
NBUF = 2
DUNROLL = 6


def _affine_gate(x, idx, gb):
    mesh = plsc.VectorSubcoreMesh(core_axis_name="c", subcore_axis_name="s")

    scratch = [
        pltpu.VMEM((NBUF, C, D), jnp.float32),
        pltpu.VMEM((8, D), jnp.int32),
        pltpu.VMEM((TPW,), jnp.int32),
    ]
    scratch += [pltpu.SemaphoreType.DMA] * (2 * NBUF)

    @functools.partial(
        pl.kernel,
        mesh=mesh,
        out_type=jax.ShapeDtypeStruct((N_TOK, D), jnp.float32),
        scratch_types=scratch,
    )
    def k(x_hbm, idx_hbm, gb_hbm, out_hbm, xb, tab, ids, *sems):
        s_x = sems[0:NBUF]
        s_o = sems[NBUF:2 * NBUF]
        wid = lax.axis_index("s") * NC + lax.axis_index("c")
        wbase = wid * TPW

        def start_in(b, ci):
            base = wbase + ci * C
            pltpu.make_async_copy(
                x_hbm.at[pl.ds(base, C)], xb.at[b], s_x[b]).start()

        def wait_in(b):
            pltpu.make_async_copy(
                x_hbm.at[pl.ds(0, C)], xb.at[b], s_x[b]).wait()

        def start_out(b, ci):
            base = wbase + ci * C
            pltpu.make_async_copy(
                xb.at[b], out_hbm.at[pl.ds(base, C)], s_o[b]).start()

        def wait_out(b):
            pltpu.make_async_copy(
                xb.at[b], out_hbm.at[pl.ds(0, C)], s_o[b]).wait()

        pltpu.sync_copy(gb_hbm, tab)
        pltpu.sync_copy(idx_hbm.at[pl.ds(wbase, TPW)], ids)

        for b in range(NBUF):
            start_in(b, b)

        def compute(b, ci):
            def tg_body(tg, tc):
                ev = ids[pl.ds(ci * C + tg * LANES, LANES)]
                for j in range(LANES):
                    e = ev[j]
                    t = tg * LANES + j

                    @plsc.parallel_loop(0, D // LANES, unroll=DUNROLL)
                    def d_body(d):
                        sl = pl.ds(d * LANES, LANES)
                        w = tab[e, sl]
                        g = lax.bitcast_convert_type(
                            w & jnp.int32(-65536), jnp.float32)
                        bta = lax.bitcast_convert_type(
                            lax.shift_left(w, 16), jnp.float32)
                        xb[b, t, sl] = xb[b, t, sl] * g + bta
                return tc

            lax.fori_loop(0, C // LANES, tg_body, 0)

        def group_body(g0, carry):
            for b in range(NBUF):
                ci = g0 * NBUF + b
                wait_in(b)
                compute(b, ci)
                start_out(b, ci)
                bp = (b - 1) % NBUF
                cip = ci - 1 + NBUF

                @pl.when(jnp.logical_and(ci >= 1, cip < NCHUNK))
                def _():
                    wait_out(bp)
                    start_in(bp, cip)

            return carry

        lax.fori_loop(0, NCHUNK // NBUF, group_body, 0)

        for b in range(NBUF):
            wait_out(b)

    return k(x, idx, gb)


def kernel(x, group, gamma, beta):
    idx = group.reshape(-1)
    gbits = jax.lax.bitcast_convert_type(
        gamma.astype(jnp.bfloat16), jnp.uint16).astype(jnp.uint32)
    bbits = jax.lax.bitcast_convert_type(
        beta.astype(jnp.bfloat16), jnp.uint16).astype(jnp.uint32)
    gb = jax.lax.bitcast_convert_type((gbits << 16) | bbits, jnp.int32)
    return _affine_gate(x, idx, gb)


# speedup vs baseline: 2.8305x; 1.1908x over previous
import functools

import jax
import jax.numpy as jnp
from jax import lax
from jax.experimental import pallas as pl
from jax.experimental.pallas import tpu as pltpu
from jax.experimental.pallas import tpu_sc as plsc

N_TOK = 32768
D = 768
LANES = 16
NC = 2
NS = 16
NW = NC * NS
TPW = N_TOK // NW
C = 64
NCHUNK = TPW // C
NBUF = 2
DUNROLL = 4


def _affine_gate(x, idx, gb):
    mesh = plsc.VectorSubcoreMesh(core_axis_name="c", subcore_axis_name="s")

    scratch = [
        pltpu.VMEM((NBUF, C, D), jnp.float32),
        pltpu.VMEM((8, D), jnp.int32),
        pltpu.VMEM((TPW,), jnp.int32),
    ]
    scratch += [pltpu.SemaphoreType.DMA] * (2 * NBUF)

    @functools.partial(
        pl.kernel,
        mesh=mesh,
        out_type=jax.ShapeDtypeStruct((N_TOK, D), jnp.float32),
        scratch_types=scratch,
    )
    def k(x_hbm, idx_hbm, gb_hbm, out_hbm, xb, tab, ids, *sems):
        s_x = sems[0:NBUF]
        s_o = sems[NBUF:2 * NBUF]
        wid = lax.axis_index("s") * NC + lax.axis_index("c")
        wbase = wid * TPW

        def start_in(b, ci):
            base = wbase + ci * C
            pltpu.make_async_copy(
                x_hbm.at[pl.ds(base, C)], xb.at[b], s_x[b]).start()

        def wait_in(b):
            pltpu.make_async_copy(
                x_hbm.at[pl.ds(0, C)], xb.at[b], s_x[b]).wait()

        def start_out(b, ci):
            base = wbase + ci * C
            pltpu.make_async_copy(
                xb.at[b], out_hbm.at[pl.ds(base, C)], s_o[b]).start()

        def wait_out(b):
            pltpu.make_async_copy(
                xb.at[b], out_hbm.at[pl.ds(0, C)], s_o[b]).wait()

        pltpu.sync_copy(gb_hbm, tab)
        pltpu.sync_copy(idx_hbm.at[pl.ds(wbase, TPW)], ids)

        for b in range(NBUF):
            start_in(b, b)

        def compute(b, ci):
            def tg_body(tg, tc):
                ev = ids[pl.ds(ci * C + tg * LANES, LANES)]
                es = [ev[j] for j in range(LANES)]

                @plsc.parallel_loop(0, D // LANES, unroll=DUNROLL)
                def d_body(d):
                    sl = pl.ds(d * LANES, LANES)
                    for j in range(LANES):
                        t = tg * LANES + j
                        w = tab[es[j], sl]
                        g = lax.bitcast_convert_type(
                            w & jnp.int32(-65536), jnp.float32)
                        bta = lax.bitcast_convert_type(
                            lax.shift_left(w, 16), jnp.float32)
                        xb[b, t, sl] = xb[b, t, sl] * g + bta
                return tc

            lax.fori_loop(0, C // LANES, tg_body, 0)

        def group_body(g0, carry):
            for b in range(NBUF):
                ci = g0 * NBUF + b
                wait_in(b)
                compute(b, ci)
                start_out(b, ci)
                bp = (b - 1) % NBUF
                cip = ci - 1 + NBUF

                @pl.when(jnp.logical_and(ci >= 1, cip < NCHUNK))
                def _():
                    wait_out(bp)
                    start_in(bp, cip)

            return carry

        lax.fori_loop(0, NCHUNK // NBUF, group_body, 0)

        for b in range(NBUF):
            wait_out(b)

    return k(x, idx, gb)


def kernel(x, group, gamma, beta):
    idx = group.reshape(-1)
    gbits = jax.lax.bitcast_convert_type(
        gamma.astype(jnp.bfloat16), jnp.uint16).astype(jnp.uint32)
    bbits = jax.lax.bitcast_convert_type(
        beta.astype(jnp.bfloat16), jnp.uint16).astype(jnp.uint32)
    gb = jax.lax.bitcast_convert_type((gbits << 16) | bbits, jnp.int32)
    return _affine_gate(x, idx, gb)


# speedup vs baseline: 4.1398x; 1.4626x over previous
import functools

import jax
import jax.numpy as jnp
from jax import lax
from jax.experimental import pallas as pl
from jax.experimental.pallas import tpu as pltpu
from jax.experimental.pallas import tpu_sc as plsc

N_TOK = 32768
D = 768
LANES = 16
NC = 2
NS = 16
NW = NC * NS
TPW = N_TOK // NW
C = 32
NCHUNK = TPW // C
NBUF = 4
DUNROLL = 4


def _affine_gate(x, idx, gb):
    mesh = plsc.VectorSubcoreMesh(core_axis_name="c", subcore_axis_name="s")

    scratch = [
        pltpu.VMEM((NBUF, C, D), jnp.float32),
        pltpu.VMEM((8, D), jnp.int32),
        pltpu.VMEM((TPW,), jnp.int32),
    ]
    scratch += [pltpu.SemaphoreType.DMA] * (2 * NBUF)

    @functools.partial(
        pl.kernel,
        mesh=mesh,
        out_type=jax.ShapeDtypeStruct((N_TOK, D), jnp.float32),
        scratch_types=scratch,
    )
    def k(x_hbm, idx_hbm, gb_hbm, out_hbm, xb, tab, ids, *sems):
        s_x = sems[0:NBUF]
        s_o = sems[NBUF:2 * NBUF]
        wid = lax.axis_index("s") * NC + lax.axis_index("c")
        wbase = wid * TPW

        def start_in(b, ci):
            base = wbase + ci * C
            pltpu.make_async_copy(
                x_hbm.at[pl.ds(base, C)], xb.at[b], s_x[b]).start()

        def wait_in(b):
            pltpu.make_async_copy(
                x_hbm.at[pl.ds(0, C)], xb.at[b], s_x[b]).wait()

        def start_out(b, ci):
            base = wbase + ci * C
            pltpu.make_async_copy(
                xb.at[b], out_hbm.at[pl.ds(base, C)], s_o[b]).start()

        def wait_out(b):
            pltpu.make_async_copy(
                xb.at[b], out_hbm.at[pl.ds(0, C)], s_o[b]).wait()

        pltpu.sync_copy(gb_hbm, tab)
        pltpu.sync_copy(idx_hbm.at[pl.ds(wbase, TPW)], ids)

        for b in range(NBUF):
            start_in(b, b)

        def compute(b, ci):
            def tg_body(tg, tc):
                ev = ids[pl.ds(ci * C + tg * LANES, LANES)]
                es = [ev[j] for j in range(LANES)]

                @plsc.parallel_loop(0, D // LANES, unroll=DUNROLL)
                def d_body(d):
                    sl = pl.ds(d * LANES, LANES)
                    for j in range(LANES):
                        t = tg * LANES + j
                        w = tab[es[j], sl]
                        g = lax.bitcast_convert_type(
                            w & jnp.int32(-65536), jnp.float32)
                        bta = lax.bitcast_convert_type(
                            lax.shift_left(w, 16), jnp.float32)
                        xb[b, t, sl] = xb[b, t, sl] * g + bta
                return tc

            lax.fori_loop(0, C // LANES, tg_body, 0)

        def group_body(g0, carry):
            for b in range(NBUF):
                ci = g0 * NBUF + b
                wait_in(b)
                compute(b, ci)
                start_out(b, ci)
                bp = (b - 1) % NBUF
                cip = ci - 1 + NBUF

                @pl.when(jnp.logical_and(ci >= 1, cip < NCHUNK))
                def _():
                    wait_out(bp)
                    start_in(bp, cip)

            return carry

        lax.fori_loop(0, NCHUNK // NBUF, group_body, 0)

        for b in range(NBUF):
            wait_out(b)

    return k(x, idx, gb)


def kernel(x, group, gamma, beta):
    idx = group.reshape(-1)
    gbits = jax.lax.bitcast_convert_type(
        gamma.astype(jnp.bfloat16), jnp.uint16).astype(jnp.uint32)
    bbits = jax.lax.bitcast_convert_type(
        beta.astype(jnp.bfloat16), jnp.uint16).astype(jnp.uint32)
    gb = jax.lax.bitcast_convert_type((gbits << 16) | bbits, jnp.int32)
    return _affine_gate(x, idx, gb)
